# baseline (speedup 1.0000x reference)
import functools
import math

import jax
import jax.numpy as jnp
from jax import lax
from jax.experimental import pallas as pl
from jax.experimental.pallas import tpu as pltpu
from jax.experimental.pallas import tpu_sc as plsc

DIM = 128
MAX_LEN = 200
VOCAB = 15
VOCAB_PAD = 16
SEQ_LEN = 200
BATCH = 4096
SCALE = math.sqrt(DIM)

NUM_CORES = 2
NUM_SUBCORES = 16
NUM_WORKERS = NUM_CORES * NUM_SUBCORES

FLAT = BATCH * SEQ_LEN
PER_WORKER = FLAT // NUM_WORKERS
K = 128
N_STEPS = PER_WORKER // K
NBUF = 4

TAB_ROWS = MAX_LEN * VOCAB_PAD
TAB_PER_SUB = TAB_ROWS // NUM_SUBCORES
TAB_CHUNKS = ((0, 104), (104, 96))
POS_STAGE = 13


def _sc_body(x_hbm, tok_hbm, pos_hbm, out_hbm, xv, tokv, posv, shared_tab,
             *bufs):
    idx = bufs[0:NBUF]
    rows = bufs[NBUF:2 * NBUF]
    gsem = bufs[2 * NBUF:3 * NBUF]
    ssem = bufs[3 * NBUF:4 * NBUF]
    xsem = bufs[4 * NBUF]

    sid = lax.axis_index("s")
    wid = sid * NUM_CORES + lax.axis_index("c")
    base = wid * PER_WORKER

    pltpu.async_copy(x_hbm.at[pl.ds(base, PER_WORKER)], xv, xsem)

    base_r = sid * TAB_PER_SUB
    p_start = base_r // VOCAB_PAD
    pltpu.sync_copy(tok_hbm, tokv)
    pltpu.sync_copy(pos_hbm.at[pl.ds(p_start * DIM, POS_STAGE * DIM)], posv)

    def build_row(r_local, chunk_base):
        r = base_r + chunk_base + r_local
        v = lax.rem(r, jnp.int32(VOCAB_PAD))
        p = r // VOCAB_PAD - p_start
        for dd in range(DIM // 16):
            tv = tokv[pl.ds(v * DIM + dd * 16, 16)]
            pv = posv[pl.ds(p * DIM + dd * 16, 16)]
            rows[0][r_local, pl.ds(dd * 16, 16)] = tv * SCALE + pv
        return chunk_base

    for c_off, c_len in TAB_CHUNKS:
        lax.fori_loop(0, c_len, build_row, jnp.int32(c_off))
        pltpu.sync_copy(rows[0].at[pl.ds(0, c_len)],
                        shared_tab.at[pl.ds(base_r + c_off, c_len)])

    pltpu.make_async_copy(x_hbm.at[pl.ds(base, PER_WORKER)], xv, xsem).wait()
    plsc.subcore_barrier()

    iota16 = lax.iota(jnp.int32, 16)

    def fill_idx(b, step):
        for jj in range(K // 16):
            n = step * K + jj * 16 + iota16
            spos = lax.rem(n, jnp.int32(SEQ_LEN))
            xval = xv[pl.ds(step * K + jj * 16, 16)]
            idx[b][pl.ds(jj * 16, 16)] = spos * VOCAB_PAD + xval

    def start_gather(b):
        pltpu.async_copy(shared_tab.at[idx[b]], rows[b], gsem[b])

    def wait_gather(b):
        pltpu.make_async_copy(shared_tab.at[idx[b]], rows[b], gsem[b]).wait()

    def start_store(b, step):
        pltpu.async_copy(rows[b], out_hbm.at[pl.ds(base + step * K, K)], ssem[b])

    def wait_store(b, step):
        pltpu.make_async_copy(
            rows[b], out_hbm.at[pl.ds(base + step * K, K)], ssem[b]).wait()

    for b in range(NBUF):
        fill_idx(b, jnp.int32(b))
        start_gather(b)

    def group(g, carry):
        for b in range(NBUF):
            s = g * NBUF + b
            wait_gather(b)
            start_store(b, s)
        for b in range(NBUF):
            s = g * NBUF + b
            wait_store(b, s)
            fill_idx(b, s + NBUF)
            start_gather(b)
        return carry

    n_groups = N_STEPS // NBUF
    lax.fori_loop(0, n_groups - 1, group, jnp.int32(0))

    g = jnp.int32(n_groups - 1)
    for b in range(NBUF):
        s = g * NBUF + b
        wait_gather(b)
        start_store(b, s)
    for b in range(NBUF):
        s = g * NBUF + b
        wait_store(b, s)


def _sc_run(x_flat, tok_pad, pos):
    mesh = plsc.VectorSubcoreMesh(core_axis_name="c", subcore_axis_name="s",
                                  num_cores=NUM_CORES)
    scratch = (
        [pltpu.VMEM((PER_WORKER,), jnp.int32)]
        + [pltpu.VMEM((VOCAB_PAD * DIM,), jnp.float32)]
        + [pltpu.VMEM((POS_STAGE * DIM,), jnp.float32)]
        + [pltpu.VMEM_SHARED((TAB_ROWS, DIM), jnp.float32)]
        + [pltpu.VMEM((K,), jnp.int32) for _ in range(NBUF)]
        + [pltpu.VMEM((K, DIM), jnp.float32) for _ in range(NBUF)]
        + [pltpu.SemaphoreType.DMA for _ in range(2 * NBUF + 1)]
    )
    run = functools.partial(
        pl.kernel,
        mesh=mesh,
        out_type=jax.ShapeDtypeStruct((FLAT, DIM), jnp.float32),
        scratch_types=scratch,
    )(_sc_body)
    return run(x_flat, tok_pad, pos)


def kernel(x, token_embedding, position_embedding):
    tok_pad = jnp.pad(token_embedding, ((0, VOCAB_PAD - VOCAB), (0, 0)))
    out_flat = _sc_run(x.reshape(FLAT), tok_pad.reshape(VOCAB_PAD * DIM),
                       position_embedding.reshape(MAX_LEN * DIM))
    return out_flat.reshape(BATCH, SEQ_LEN, DIM)


# speedup vs baseline: 1.0501x; 1.0501x over previous
import functools
import math

import jax
import jax.numpy as jnp
from jax import lax
from jax.experimental import pallas as pl
from jax.experimental.pallas import tpu as pltpu
from jax.experimental.pallas import tpu_sc as plsc

DIM = 128
MAX_LEN = 200
VOCAB = 15
VOCAB_PAD = 16
SEQ_LEN = 200
BATCH = 4096
SCALE = math.sqrt(DIM)

NUM_CORES = 2
NUM_SUBCORES = 16
NUM_WORKERS = NUM_CORES * NUM_SUBCORES

FLAT = BATCH * SEQ_LEN
PER_WORKER = FLAT // NUM_WORKERS
K = 128
N_STEPS = PER_WORKER // K
NBUF = 4

TAB_ROWS = MAX_LEN * VOCAB_PAD


def _table_body(tok_ref, pos_ref, out_ref):
    tok = tok_ref[...] * SCALE
    pos = pos_ref[...]
    out_ref[...] = tok[None, :, :] + pos[:, None, :]


def _build_table(tok_pad, pos):
    return pl.pallas_call(
        _table_body,
        out_shape=jax.ShapeDtypeStruct((MAX_LEN, VOCAB_PAD, DIM), jnp.float32),
    )(tok_pad, pos)


def _sc_body(x_hbm, fused_hbm, out_hbm, xv, shared_tab, *bufs):
    idx = bufs[0:NBUF]
    rows = bufs[NBUF:2 * NBUF]
    gsem = bufs[2 * NBUF:3 * NBUF]
    ssem = bufs[3 * NBUF:4 * NBUF]
    xsem = bufs[4 * NBUF]

    sid = lax.axis_index("s")
    wid = sid * NUM_CORES + lax.axis_index("c")
    base = wid * PER_WORKER

    pltpu.async_copy(x_hbm.at[pl.ds(base, PER_WORKER)], xv, xsem)

    tab_rows = TAB_ROWS // NUM_SUBCORES
    pltpu.sync_copy(fused_hbm.at[pl.ds(sid * tab_rows, tab_rows)],
                    shared_tab.at[pl.ds(sid * tab_rows, tab_rows)])

    pltpu.make_async_copy(x_hbm.at[pl.ds(base, PER_WORKER)], xv, xsem).wait()
    plsc.subcore_barrier()

    iota16 = lax.iota(jnp.int32, 16)

    def fill_idx(b, step):
        for jj in range(K // 16):
            n = step * K + jj * 16 + iota16
            spos = lax.rem(n, jnp.int32(SEQ_LEN))
            xval = xv[pl.ds(step * K + jj * 16, 16)]
            idx[b][pl.ds(jj * 16, 16)] = spos * VOCAB_PAD + xval

    def start_gather(b):
        pltpu.async_copy(shared_tab.at[idx[b]], rows[b], gsem[b])

    def wait_gather(b):
        pltpu.make_async_copy(shared_tab.at[idx[b]], rows[b], gsem[b]).wait()

    def start_store(b, step):
        pltpu.async_copy(rows[b], out_hbm.at[pl.ds(base + step * K, K)], ssem[b])

    def wait_store(b, step):
        pltpu.make_async_copy(
            rows[b], out_hbm.at[pl.ds(base + step * K, K)], ssem[b]).wait()

    for b in range(NBUF):
        fill_idx(b, jnp.int32(b))
        start_gather(b)

    def group(g, carry):
        for b in range(NBUF):
            s = g * NBUF + b
            wait_gather(b)
            start_store(b, s)
        for b in range(NBUF):
            s = g * NBUF + b
            wait_store(b, s)
            fill_idx(b, s + NBUF)
            start_gather(b)
        return carry

    n_groups = N_STEPS // NBUF
    lax.fori_loop(0, n_groups - 1, group, jnp.int32(0))

    g = jnp.int32(n_groups - 1)
    for b in range(NBUF):
        s = g * NBUF + b
        wait_gather(b)
        start_store(b, s)
    for b in range(NBUF):
        s = g * NBUF + b
        wait_store(b, s)


def _sc_gather(x_flat, fused_flat):
    mesh = plsc.VectorSubcoreMesh(core_axis_name="c", subcore_axis_name="s",
                                  num_cores=NUM_CORES)
    scratch = (
        [pltpu.VMEM((PER_WORKER,), jnp.int32)]
        + [pltpu.VMEM_SHARED((TAB_ROWS, DIM), jnp.float32)]
        + [pltpu.VMEM((K,), jnp.int32) for _ in range(NBUF)]
        + [pltpu.VMEM((K, DIM), jnp.float32) for _ in range(NBUF)]
        + [pltpu.SemaphoreType.DMA for _ in range(2 * NBUF + 1)]
    )
    run = functools.partial(
        pl.kernel,
        mesh=mesh,
        out_type=jax.ShapeDtypeStruct((FLAT, DIM), jnp.float32),
        scratch_types=scratch,
    )(_sc_body)
    return run(x_flat, fused_flat)


def kernel(x, token_embedding, position_embedding):
    tok_pad = jnp.pad(token_embedding, ((0, VOCAB_PAD - VOCAB), (0, 0)))
    fused = _build_table(tok_pad, position_embedding)
    fused_flat = fused.reshape(TAB_ROWS, DIM)
    out_flat = _sc_gather(x.reshape(FLAT), fused_flat)
    return out_flat.reshape(BATCH, SEQ_LEN, DIM)


# speedup vs baseline: 1.0604x; 1.0098x over previous
import functools
import math

import jax
import jax.numpy as jnp
from jax import lax
from jax.experimental import pallas as pl
from jax.experimental.pallas import tpu as pltpu
from jax.experimental.pallas import tpu_sc as plsc

DIM = 128
MAX_LEN = 200
VOCAB = 15
VOCAB_PAD = 16
SEQ_LEN = 200
BATCH = 4096
SCALE = math.sqrt(DIM)

NUM_CORES = 2
NUM_SUBCORES = 16
NUM_WORKERS = NUM_CORES * NUM_SUBCORES

FLAT = BATCH * SEQ_LEN
PER_WORKER = FLAT // NUM_WORKERS
K = 64
N_STEPS = PER_WORKER // K
NBUF = 8

TAB_ROWS = MAX_LEN * VOCAB_PAD


def _table_body(tok_ref, pos_ref, out_ref):
    tok = tok_ref[...] * SCALE
    pos = pos_ref[...]
    out_ref[...] = tok[None, :, :] + pos[:, None, :]


def _build_table(tok_pad, pos):
    return pl.pallas_call(
        _table_body,
        out_shape=jax.ShapeDtypeStruct((MAX_LEN, VOCAB_PAD, DIM), jnp.float32),
    )(tok_pad, pos)


def _sc_body(x_hbm, fused_hbm, out_hbm, xv, shared_tab, *bufs):
    idx = bufs[0:NBUF]
    rows = bufs[NBUF:2 * NBUF]
    gsem = bufs[2 * NBUF:3 * NBUF]
    ssem = bufs[3 * NBUF:4 * NBUF]
    xsem = bufs[4 * NBUF]

    sid = lax.axis_index("s")
    wid = sid * NUM_CORES + lax.axis_index("c")
    base = wid * PER_WORKER

    pltpu.async_copy(x_hbm.at[pl.ds(base, PER_WORKER)], xv, xsem)

    tab_rows = TAB_ROWS // NUM_SUBCORES
    pltpu.sync_copy(fused_hbm.at[pl.ds(sid * tab_rows, tab_rows)],
                    shared_tab.at[pl.ds(sid * tab_rows, tab_rows)])

    pltpu.make_async_copy(x_hbm.at[pl.ds(base, PER_WORKER)], xv, xsem).wait()
    plsc.subcore_barrier()

    iota16 = lax.iota(jnp.int32, 16)

    def fill_idx(b, step):
        for jj in range(K // 16):
            n = step * K + jj * 16 + iota16
            spos = lax.rem(n, jnp.int32(SEQ_LEN))
            xval = xv[pl.ds(step * K + jj * 16, 16)]
            idx[b][pl.ds(jj * 16, 16)] = spos * VOCAB_PAD + xval

    def start_gather(b):
        pltpu.async_copy(shared_tab.at[idx[b]], rows[b], gsem[b])

    def wait_gather(b):
        pltpu.make_async_copy(shared_tab.at[idx[b]], rows[b], gsem[b]).wait()

    def start_store(b, step):
        pltpu.async_copy(rows[b], out_hbm.at[pl.ds(base + step * K, K)], ssem[b])

    def wait_store(b, step):
        pltpu.make_async_copy(
            rows[b], out_hbm.at[pl.ds(base + step * K, K)], ssem[b]).wait()

    for b in range(NBUF):
        fill_idx(b, jnp.int32(b))
        start_gather(b)

    def group(g, carry):
        for b in range(NBUF):
            s = g * NBUF + b
            wait_gather(b)
            start_store(b, s)
        for b in range(NBUF):
            s = g * NBUF + b
            wait_store(b, s)
            fill_idx(b, s + NBUF)
            start_gather(b)
        return carry

    n_groups = N_STEPS // NBUF
    lax.fori_loop(0, n_groups - 1, group, jnp.int32(0))

    g = jnp.int32(n_groups - 1)
    for b in range(NBUF):
        s = g * NBUF + b
        wait_gather(b)
        start_store(b, s)
    for b in range(NBUF):
        s = g * NBUF + b
        wait_store(b, s)


def _sc_gather(x_flat, fused_flat):
    mesh = plsc.VectorSubcoreMesh(core_axis_name="c", subcore_axis_name="s",
                                  num_cores=NUM_CORES)
    scratch = (
        [pltpu.VMEM((PER_WORKER,), jnp.int32)]
        + [pltpu.VMEM_SHARED((TAB_ROWS, DIM), jnp.float32)]
        + [pltpu.VMEM((K,), jnp.int32) for _ in range(NBUF)]
        + [pltpu.VMEM((K, DIM), jnp.float32) for _ in range(NBUF)]
        + [pltpu.SemaphoreType.DMA for _ in range(2 * NBUF + 1)]
    )
    run = functools.partial(
        pl.kernel,
        mesh=mesh,
        out_type=jax.ShapeDtypeStruct((FLAT, DIM), jnp.float32),
        scratch_types=scratch,
    )(_sc_body)
    return run(x_flat, fused_flat)


def kernel(x, token_embedding, position_embedding):
    tok_pad = jnp.pad(token_embedding, ((0, VOCAB_PAD - VOCAB), (0, 0)))
    fused = _build_table(tok_pad, position_embedding)
    fused_flat = fused.reshape(TAB_ROWS, DIM)
    out_flat = _sc_gather(x.reshape(FLAT), fused_flat)
    return out_flat.reshape(BATCH, SEQ_LEN, DIM)
